# baseline (speedup 1.0000x reference)
import jax
import jax.numpy as jnp
from jax import lax
from jax.experimental import pallas as pl
from jax.experimental.pallas import tpu as pltpu
from jax.experimental.pallas import tpu_sc as plsc

VOCAB = 1000000
N_EMBD = 64
BATCH = 16384
SEQ = 50

_info = plsc.get_sparse_core_info()
NC = _info.num_cores
NS = _info.num_subcores
NW = NC * NS

BW_ = BATCH // NW
CB = 256
N_CHUNKS = SEQ * 2
OUT_LEN = SEQ * N_EMBD * BATCH


def _body(ids_hbm, table_hbm, out_hbm,
          idx0, idx1, g0, g1, t0, t1,
          si0, si1, sg0, sg1, sw0, sw1):
    idxb = (idx0, idx1)
    gb = (g0, g1)
    tb = (t0, t1)
    si = (si0, si1)
    sg = (sg0, sg1)
    sw = (sw0, sw1)

    wid = lax.axis_index("s") * NC + lax.axis_index("c")
    wb0 = wid * BW_

    iota = lax.iota(jnp.int32, 16)
    rowvs = [iota + (jj * 128 + ci * 16) for jj in range(2) for ci in range(8)]

    def idx_off(k):
        return (k // 2) * BATCH + wb0 + (k % 2) * CB

    def idx_start(k, b):
        pltpu.async_copy(ids_hbm.at[pl.ds(idx_off(k), CB)], idxb[b], si[b])

    def idx_wait(b):
        pltpu.make_async_copy(ids_hbm.at[pl.ds(0, CB)], idxb[b], si[b]).wait()

    def gather_start(b):
        pltpu.async_copy(table_hbm.at[idxb[b]], gb[b], sg[b])

    def gather_wait(b):
        pltpu.make_async_copy(table_hbm.at[idxb[b]], gb[b], sg[b]).wait()

    def writes_start(k, b):
        s = k // 2
        h = k % 2
        for I in range(8):
            off = ((s * 8 + I) * 128 + wid * 4 + h * 2) * 1024
            pltpu.async_copy(tb[b].at[pl.ds(I * 2048, 2048)],
                             out_hbm.at[pl.ds(off, 2048)], sw[b])

    def writes_wait(b):
        for I in range(8):
            pltpu.make_async_copy(tb[b].at[pl.ds(I * 2048, 2048)],
                                  out_hbm.at[pl.ds(0, 2048)], sw[b]).wait()

    def transpose(b):
        G = gb[b]
        T = tb[b]

        def e_step(e, carry):
            evec = lax.broadcast_in_dim(e, (16,), ())
            tbase = (e >> 3) * 2048 + (e & 7) * 128
            for jj in range(2):
                for ci in range(8):
                    v = plsc.load_gather(G, [rowvs[jj * 8 + ci], evec])
                    T[pl.ds(tbase + jj * 1024 + ci * 16, 16)] = v
            return carry

        lax.fori_loop(0, N_EMBD, e_step, 0, unroll=False)

    idx_start(0, 0)
    idx_start(1, 1)
    idx_wait(0)
    gather_start(0)

    def chunk(k, b):
        gather_wait(b)

        nb = b ^ 1

        @pl.when(k + 1 < N_CHUNKS)
        def _():
            idx_wait(nb)
            gather_start(nb)

        @pl.when(k + 2 < N_CHUNKS)
        def _():
            idx_start(k + 2, b)

        @pl.when(k >= 2)
        def _():
            writes_wait(b)

        transpose(b)
        writes_start(k, b)

    def outer(o, carry):
        k = o * 2
        chunk(k, 0)
        chunk(k + 1, 1)
        return carry

    lax.fori_loop(0, N_CHUNKS // 2, outer, 0, unroll=False)

    writes_wait(0)
    writes_wait(1)


@jax.jit
def kernel(input_ids, wte):
    ids_flat = input_ids.T.reshape(-1)
    mesh = plsc.VectorSubcoreMesh(core_axis_name="c", subcore_axis_name="s")
    out1d = pl.kernel(
        _body,
        out_type=jax.ShapeDtypeStruct((OUT_LEN,), jnp.float32),
        mesh=mesh,
        scratch_types=(
            [pltpu.VMEM((CB,), jnp.int32) for _ in range(2)]
            + [pltpu.VMEM((CB, N_EMBD), jnp.float32) for _ in range(2)]
            + [pltpu.VMEM((8 * 2048,), jnp.float32) for _ in range(2)]
            + [pltpu.SemaphoreType.DMA for _ in range(6)]
        ),
        compiler_params=pltpu.CompilerParams(use_tc_tiling_on_sc=False,
                                             needs_layout_passes=False),
    )(ids_flat, wte)
    X = out1d.reshape(SEQ, 8, 128, 8, 128)
    return X.transpose(2, 4, 0, 1, 3).reshape(BATCH, SEQ, N_EMBD)


# speedup vs baseline: 1.3274x; 1.3274x over previous
import jax
import jax.numpy as jnp
from jax import lax
from jax.experimental import pallas as pl
from jax.experimental.pallas import tpu as pltpu
from jax.experimental.pallas import tpu_sc as plsc

VOCAB = 1000000
N_EMBD = 64
BATCH = 16384
SEQ = 50

_info = plsc.get_sparse_core_info()
NC = _info.num_cores
NS = _info.num_subcores
NW = NC * NS

BW_ = BATCH // NW
CB = 256
N_CHUNKS = SEQ * 2
OUT_LEN = SEQ * N_EMBD * BATCH


def _body(ids_hbm, table_hbm, out_hbm,
          idx0, idx1, g0, g1, t0, t1,
          si0, si1, sg0, sg1, sw0, sw1):
    idxb = (idx0, idx1)
    gb = (g0, g1)
    tb = (t0, t1)
    si = (si0, si1)
    sg = (sg0, sg1)
    sw = (sw0, sw1)

    wid = lax.axis_index("s") * NC + lax.axis_index("c")
    wb0 = wid * BW_

    iota = lax.iota(jnp.int32, 16)
    rowvs = [iota + (jj * 128 + ci * 16) for jj in range(2) for ci in range(8)]

    def idx_off(k):
        return (k // 2) * BATCH + wb0 + (k % 2) * CB

    def idx_start(k, b):
        pltpu.async_copy(ids_hbm.at[pl.ds(idx_off(k), CB)], idxb[b], si[b])

    def idx_wait(b):
        pltpu.make_async_copy(ids_hbm.at[pl.ds(0, CB)], idxb[b], si[b]).wait()

    def gather_start(b):
        pltpu.async_copy(table_hbm.at[idxb[b]], gb[b], sg[b])

    def gather_wait(b):
        pltpu.make_async_copy(table_hbm.at[idxb[b]], gb[b], sg[b]).wait()

    def writes_start(k, b):
        s = k // 2
        h = k % 2
        for I in range(8):
            off = ((s * 8 + I) * 128 + wid * 4 + h * 2) * 1024
            pltpu.async_copy(tb[b].at[pl.ds(I * 2048, 2048)],
                             out_hbm.at[pl.ds(off, 2048)], sw[b])

    def writes_wait(b):
        for I in range(8):
            pltpu.make_async_copy(tb[b].at[pl.ds(I * 2048, 2048)],
                                  out_hbm.at[pl.ds(0, 2048)], sw[b]).wait()

    def transpose(b):
        G = gb[b]
        T = tb[b]

        @plsc.parallel_loop(0, N_EMBD, 1, unroll=8)
        def _(e):
            evec = lax.broadcast_in_dim(e, (16,), ())
            tbase = (e >> 3) * 2048 + (e & 7) * 128
            for jj in range(2):
                for ci in range(8):
                    v = plsc.load_gather(G, [rowvs[jj * 8 + ci], evec])
                    T[pl.ds(tbase + jj * 1024 + ci * 16, 16)] = v

    idx_start(0, 0)
    idx_start(1, 1)
    idx_wait(0)
    gather_start(0)

    def chunk(k, b):
        gather_wait(b)

        nb = b ^ 1

        @pl.when(k + 1 < N_CHUNKS)
        def _():
            idx_wait(nb)
            gather_start(nb)

        @pl.when(k + 2 < N_CHUNKS)
        def _():
            idx_start(k + 2, b)

        @pl.when(k >= 2)
        def _():
            writes_wait(b)

        transpose(b)
        writes_start(k, b)

    def outer(o, carry):
        k = o * 2
        chunk(k, 0)
        chunk(k + 1, 1)
        return carry

    lax.fori_loop(0, N_CHUNKS // 2, outer, 0, unroll=False)

    writes_wait(0)
    writes_wait(1)


@jax.jit
def kernel(input_ids, wte):
    ids_flat = input_ids.T.reshape(-1)
    mesh = plsc.VectorSubcoreMesh(core_axis_name="c", subcore_axis_name="s")
    out1d = pl.kernel(
        _body,
        out_type=jax.ShapeDtypeStruct((OUT_LEN,), jnp.float32),
        mesh=mesh,
        scratch_types=(
            [pltpu.VMEM((CB,), jnp.int32) for _ in range(2)]
            + [pltpu.VMEM((CB, N_EMBD), jnp.float32) for _ in range(2)]
            + [pltpu.VMEM((8 * 2048,), jnp.float32) for _ in range(2)]
            + [pltpu.SemaphoreType.DMA for _ in range(6)]
        ),
        compiler_params=pltpu.CompilerParams(use_tc_tiling_on_sc=False,
                                             needs_layout_passes=False),
    )(ids_flat, wte)
    X = out1d.reshape(SEQ, 8, 128, 8, 128)
    return X.transpose(2, 4, 0, 1, 3).reshape(BATCH, SEQ, N_EMBD)


# speedup vs baseline: 2.1911x; 1.6507x over previous
import jax
import jax.numpy as jnp
from jax import lax
from jax.experimental import pallas as pl
from jax.experimental.pallas import tpu as pltpu
from jax.experimental.pallas import tpu_sc as plsc

VOCAB = 1000000
N_EMBD = 64
BATCH = 16384
SEQ = 50

_info = plsc.get_sparse_core_info()
NC = _info.num_cores
NS = _info.num_subcores
NW = NC * NS

BW_ = BATCH // NW
CB = 128
TPW = BW_ // CB
N_CHUNKS = SEQ * TPW
N_BLOCKS = SEQ * 8 * (BATCH // 128)


def _body(ids_hbm, table_hbm, out_hbm,
          idx0, idx1, g0, g1, t0, t1,
          si0, si1, sg0, sg1, sw0, sw1):
    idxb = (idx0, idx1)
    gb = (g0, g1)
    tb = (t0, t1)
    si = (si0, si1)
    sg = (sg0, sg1)
    sw = (sw0, sw1)

    wid = lax.axis_index("s") * NC + lax.axis_index("c")
    wb0 = wid * BW_

    iota = lax.iota(jnp.int32, 16)
    r_vec = iota & 7
    i_vecs = [(iota >> 3) + 2 * e0 for e0 in range(4)]

    def idx_off(k):
        return (k // TPW) * BATCH + wb0 + (k % TPW) * CB

    def idx_start(k, b):
        pltpu.async_copy(ids_hbm.at[pl.ds(idx_off(k), CB)], idxb[b], si[b])

    def idx_wait(b):
        pltpu.make_async_copy(ids_hbm.at[pl.ds(0, CB)], idxb[b], si[b]).wait()

    def gather_start(b):
        pltpu.async_copy(table_hbm.at[idxb[b]], gb[b], sg[b])

    def gather_wait(b):
        pltpu.make_async_copy(table_hbm.at[idxb[b]], gb[b], sg[b]).wait()

    def writes_start(k, b):
        s = k // TPW
        j = wid * TPW + (k % TPW)
        for i in range(8):
            blk = (s * 8 + i) * 128 + j
            pltpu.async_copy(tb[b].at[i, :, pl.ds(0, 128)],
                             out_hbm.at[blk], sw[b])

    def writes_wait(b):
        for i in range(8):
            pltpu.make_async_copy(tb[b].at[i, :, pl.ds(0, 128)],
                                  out_hbm.at[0], sw[b]).wait()

    def transpose(b):
        G = gb[b]
        T = tb[b]

        @plsc.parallel_loop(0, CB, 1, unroll=4)
        def _(t):
            t_vec = lax.broadcast_in_dim(t, (16,), ())
            for e0 in range(4):
                v = G[t, pl.ds(e0 * 16, 16)]
                plsc.store_scatter(T, [i_vecs[e0], r_vec, t_vec], v)

    idx_start(0, 0)
    idx_start(1, 1)
    idx_wait(0)
    gather_start(0)

    def chunk(k, b):
        gather_wait(b)

        nb = b ^ 1

        @pl.when(k + 1 < N_CHUNKS)
        def _():
            idx_wait(nb)
            gather_start(nb)

        @pl.when(k + 2 < N_CHUNKS)
        def _():
            idx_start(k + 2, b)

        @pl.when(k >= 2)
        def _():
            writes_wait(b)

        transpose(b)
        writes_start(k, b)

    def outer(o, carry):
        k = o * 2
        chunk(k, 0)
        chunk(k + 1, 1)
        return carry

    lax.fori_loop(0, N_CHUNKS // 2, outer, 0, unroll=False)

    writes_wait(0)
    writes_wait(1)


@jax.jit
def kernel(input_ids, wte):
    ids_flat = input_ids.T.reshape(-1)
    mesh = plsc.VectorSubcoreMesh(core_axis_name="c", subcore_axis_name="s")
    out3 = pl.kernel(
        _body,
        out_type=jax.ShapeDtypeStruct((N_BLOCKS, 8, 128), jnp.float32),
        mesh=mesh,
        scratch_types=(
            [pltpu.VMEM((CB,), jnp.int32) for _ in range(2)]
            + [pltpu.VMEM((CB, N_EMBD), jnp.float32) for _ in range(2)]
            + [pltpu.VMEM((8, 8, 129), jnp.float32) for _ in range(2)]
            + [pltpu.SemaphoreType.DMA for _ in range(6)]
        ),
        compiler_params=pltpu.CompilerParams(use_tc_tiling_on_sc=False,
                                             needs_layout_passes=False),
    )(ids_flat, wte)
    X = out3.reshape(SEQ, 8, 128, 8, 128)
    return X.transpose(2, 4, 0, 1, 3).reshape(BATCH, SEQ, N_EMBD)
